# baseline (speedup 1.0000x reference)
import functools
import math

import jax
import jax.numpy as jnp
from jax import lax
from jax.experimental import pallas as pl
from jax.experimental.pallas import tpu as pltpu
from jax.experimental.pallas import tpu_sc as plsc

VOCAB = 1000000
D = 64
T = 200
SCALE = math.sqrt(D)
B = 4096

NC = 2
NS = 16
NW = NC * NS
SEQ_PER_W = B // NW
LANES = 16
VPR = D // LANES
IDXW = 100

C = 2
CH_ROWS = C * T
NCHUNK = SEQ_PER_W // C
GPC = CH_ROWS // IDXW
NBUF = 3
IDX_ROWS = SEQ_PER_W * T // IDXW


def _pos_encoding():
    position = jnp.arange(0, T, dtype=jnp.float32)[:, None]
    div_term = jnp.exp(
        jnp.arange(0, D, 2, dtype=jnp.float32) * (-(math.log(10000.0) / D)))
    pe = jnp.zeros((T, D), dtype=jnp.float32)
    pe = pe.at[:, 0::2].set(jnp.sin(position * div_term))
    pe = pe.at[:, 1::2].set(jnp.cos(position * div_term))
    return pe


@functools.partial(
    pl.kernel,
    mesh=plsc.VectorSubcoreMesh(core_axis_name="c", subcore_axis_name="s"),
    out_type=jax.ShapeDtypeStruct((B * T, D), jnp.float32),
    scratch_types=[
        pltpu.VMEM((T, D), jnp.float32),
        pltpu.VMEM((IDX_ROWS, IDXW), jnp.int32),
        pltpu.VMEM((NBUF, CH_ROWS, D), jnp.float32),
        pltpu.SemaphoreType.DMA((NBUF,)),
        pltpu.SemaphoreType.DMA((NBUF,)),
    ],
    compiler_params=pltpu.CompilerParams(use_tc_tiling_on_sc=False),
)
def _emb_kernel(x_hbm, tab_hbm, pe_hbm, out_hbm, pe_v, idx_v, rows_v,
                sem_g, sem_wb):
    wid = lax.axis_index("s") * NC + lax.axis_index("c")
    pltpu.sync_copy(pe_hbm, pe_v)
    pltpu.sync_copy(x_hbm.at[pl.ds(wid * IDX_ROWS, IDX_ROWS)], idx_v)

    def gather_issue(chunk, buf):
        for g in range(GPC):
            pltpu.async_copy(
                tab_hbm.at[idx_v.at[chunk * GPC + g]],
                rows_v.at[buf, pl.ds(g * IDXW, IDXW)],
                sem_g.at[buf],
            )

    def gather_wait(chunk, buf):
        for g in range(GPC):
            pltpu.make_async_copy(
                tab_hbm.at[idx_v.at[chunk * GPC + g]],
                rows_v.at[buf, pl.ds(g * IDXW, IDXW)],
                sem_g.at[buf],
            ).wait()

    def wb_issue(chunk, buf):
        base = (wid * SEQ_PER_W + chunk * C) * T
        pltpu.async_copy(
            rows_v.at[buf], out_hbm.at[pl.ds(base, CH_ROWS)], sem_wb.at[buf])

    def wb_wait(chunk, buf):
        base = (wid * SEQ_PER_W + chunk * C) * T
        pltpu.make_async_copy(
            rows_v.at[buf], out_hbm.at[pl.ds(base, CH_ROWS)],
            sem_wb.at[buf]).wait()

    gather_issue(0, 0)
    gather_issue(1, 1)

    def chunk_body(i, _):
        b = lax.rem(i, NBUF)
        gather_wait(i, b)

        def row_body(r, _):
            pes = [pe_v[r, pl.ds(d * LANES, LANES)] for d in range(VPR)]
            for s in range(C):
                row = s * T + r
                for d in range(VPR):
                    sl = pl.ds(d * LANES, LANES)
                    rows_v[b, row, sl] = rows_v[b, row, sl] * SCALE + pes[d]
            return ()

        lax.fori_loop(0, T, row_body, ())
        wb_issue(i, b)

        nb = lax.rem(i + 2, NBUF)

        @pl.when(jnp.logical_and(i >= 1, i + 2 < NCHUNK))
        def _():
            wb_wait(i - 1, nb)

        @pl.when(i + 2 < NCHUNK)
        def _():
            gather_issue(i + 2, nb)

        return ()

    lax.fori_loop(0, NCHUNK, chunk_body, ())
    for j in range(NCHUNK - 3, NCHUNK):
        wb_wait(j, j % NBUF)


def kernel(x, tok_emb):
    pe = _pos_encoding()
    x2 = x.reshape(B * T // IDXW, IDXW).astype(jnp.int32)
    out = _emb_kernel(x2, tok_emb, pe)
    return out.reshape(B, T, D)


# speedup vs baseline: 1.1772x; 1.1772x over previous
import functools
import math

import jax
import jax.numpy as jnp
from jax import lax
from jax.experimental import pallas as pl
from jax.experimental.pallas import tpu as pltpu
from jax.experimental.pallas import tpu_sc as plsc

VOCAB = 1000000
D = 64
DP = 128
T = 200
SCALE = math.sqrt(D)
B = 4096

NC = 2
NS = 16
NW = NC * NS
LANES = 16
VPR = D // LANES
CH = 128
ROWS_PER_W = B * T // NW
NCHUNK = ROWS_PER_W // CH
PE2 = T + CH
NIB = 4


def _pos_encoding():
    position = jnp.arange(0, T, dtype=jnp.float32)[:, None]
    div_term = jnp.exp(
        jnp.arange(0, D, 2, dtype=jnp.float32) * (-(math.log(10000.0) / D)))
    pe = jnp.zeros((T, D), dtype=jnp.float32)
    pe = pe.at[:, 0::2].set(jnp.sin(position * div_term))
    pe = pe.at[:, 1::2].set(jnp.cos(position * div_term))
    return pe


@functools.partial(
    pl.kernel,
    mesh=plsc.VectorSubcoreMesh(core_axis_name="c", subcore_axis_name="s"),
    out_type=jax.ShapeDtypeStruct((B, T, D), jnp.float32),
    scratch_types=[
        pltpu.VMEM((PE2, D), jnp.float32),
        pltpu.VMEM((NIB, CH), jnp.int32),
        pltpu.VMEM((2, CH, DP), jnp.float32),
        pltpu.VMEM((2, CH, D), jnp.float32),
        pltpu.SemaphoreType.DMA((NIB,)),
        pltpu.SemaphoreType.DMA((2,)),
        pltpu.SemaphoreType.DMA((2,)),
    ],
    compiler_params=pltpu.CompilerParams(use_tc_tiling_on_sc=True),
)
def _emb_kernel(x_hbm, tab_hbm, pe_hbm, out_hbm, pe_v, idx_v, rows_v, out_v,
                sem_i, sem_g, sem_wb):
    wid = lax.axis_index("s") * NC + lax.axis_index("c")
    pltpu.sync_copy(pe_hbm, pe_v)
    out_flat = out_hbm.reshape(B * T, D)
    chunk0 = wid * NCHUNK

    def idx_issue(chunk):
        pltpu.async_copy(
            x_hbm.at[chunk0 + chunk], idx_v.at[lax.rem(chunk, NIB)],
            sem_i.at[lax.rem(chunk, NIB)])

    def idx_wait(chunk):
        pltpu.make_async_copy(
            x_hbm.at[chunk0 + chunk], idx_v.at[lax.rem(chunk, NIB)],
            sem_i.at[lax.rem(chunk, NIB)]).wait()

    def gather_issue(chunk, buf):
        pltpu.async_copy(
            tab_hbm.at[idx_v.at[lax.rem(chunk, NIB)]],
            rows_v.at[buf], sem_g.at[buf])

    def gather_wait(chunk, buf):
        pltpu.make_async_copy(
            tab_hbm.at[idx_v.at[lax.rem(chunk, NIB)]],
            rows_v.at[buf], sem_g.at[buf]).wait()

    def wb_issue(chunk, buf):
        base = (chunk0 + chunk) * CH
        pltpu.async_copy(out_v.at[buf], out_flat.at[pl.ds(base, CH)],
                         sem_wb.at[buf])

    def wb_wait(chunk, buf):
        base = (chunk0 + chunk) * CH
        pltpu.make_async_copy(
            out_v.at[buf], out_flat.at[pl.ds(base, CH)],
            sem_wb.at[buf]).wait()

    for c in range(3):
        idx_issue(c)
    idx_wait(0)
    gather_issue(0, 0)
    idx_wait(1)
    gather_issue(1, 1)

    def chunk_body(i, _):
        b = lax.rem(i, 2)
        gather_wait(i, b)

        @pl.when(i >= 2)
        def _():
            wb_wait(i - 2, b)

        p0 = lax.rem((chunk0 + i) * CH, T)

        def row_body(r, _):
            for d in range(VPR):
                sl = pl.ds(d * LANES, LANES)
                out_v[b, r, sl] = rows_v[b, r, sl] * SCALE + pe_v[p0 + r, sl]
            return ()

        lax.fori_loop(0, CH, row_body, ())
        wb_issue(i, b)

        @pl.when(i + 3 < NCHUNK)
        def _():
            idx_issue(i + 3)

        @pl.when(i + 2 < NCHUNK)
        def _():
            idx_wait(i + 2)
            gather_issue(i + 2, b)

        return ()

    lax.fori_loop(0, NCHUNK, chunk_body, ())
    for j in range(NCHUNK - 2, NCHUNK):
        wb_wait(j, j % 2)


def kernel(x, tok_emb):
    pe = _pos_encoding()
    pe2 = jnp.concatenate([pe, pe[:CH]], axis=0)
    tabp = jnp.pad(tok_emb, ((0, 0), (0, DP - D)))
    x2 = x.reshape(B * T // CH, CH).astype(jnp.int32)
    return _emb_kernel(x2, tabp, pe2)


# speedup vs baseline: 1.2269x; 1.0422x over previous
import functools
import math

import jax
import jax.numpy as jnp
from jax import lax
from jax.experimental import pallas as pl
from jax.experimental.pallas import tpu as pltpu
from jax.experimental.pallas import tpu_sc as plsc

VOCAB = 1000000
D = 64
DP = 128
T = 200
SCALE = math.sqrt(D)
B = 4096

NC = 2
NS = 16
NW = NC * NS
LANES = 16
VPR = D // LANES
CH = 128
ROWS_PER_W = B * T // NW
NCHUNK = ROWS_PER_W // CH
PE2 = T + CH
NIB = 4


def _pos_encoding():
    position = jnp.arange(0, T, dtype=jnp.float32)[:, None]
    div_term = jnp.exp(
        jnp.arange(0, D, 2, dtype=jnp.float32) * (-(math.log(10000.0) / D)))
    pe = jnp.zeros((T, D), dtype=jnp.float32)
    pe = pe.at[:, 0::2].set(jnp.sin(position * div_term))
    pe = pe.at[:, 1::2].set(jnp.cos(position * div_term))
    return pe


@functools.partial(
    pl.kernel,
    mesh=plsc.VectorSubcoreMesh(core_axis_name="c", subcore_axis_name="s"),
    out_type=jax.ShapeDtypeStruct((B, T, D), jnp.float32),
    scratch_types=[
        pltpu.VMEM((PE2, D), jnp.float32),
        pltpu.VMEM((NIB, CH), jnp.int32),
        pltpu.VMEM((2, CH, DP), jnp.float32),
        pltpu.VMEM((2, CH, D), jnp.float32),
        pltpu.SemaphoreType.DMA((NIB,)),
        pltpu.SemaphoreType.DMA((2,)),
        pltpu.SemaphoreType.DMA((2,)),
    ],
    compiler_params=pltpu.CompilerParams(use_tc_tiling_on_sc=True),
)
def _emb_kernel(x_hbm, tab_hbm, pe_hbm, out_hbm, pe_v, idx_v, rows_v, out_v,
                sem_i, sem_g, sem_wb):
    wid = lax.axis_index("s") * NC + lax.axis_index("c")
    pltpu.sync_copy(pe_hbm, pe_v)
    out_flat = out_hbm.reshape(B * T, D)
    chunk0 = wid * NCHUNK

    def idx_issue(chunk):
        pltpu.async_copy(
            x_hbm.at[chunk0 + chunk], idx_v.at[lax.rem(chunk, NIB)],
            sem_i.at[lax.rem(chunk, NIB)])

    def idx_wait(chunk):
        pltpu.make_async_copy(
            x_hbm.at[chunk0 + chunk], idx_v.at[lax.rem(chunk, NIB)],
            sem_i.at[lax.rem(chunk, NIB)]).wait()

    def gather_issue(chunk, buf):
        pltpu.async_copy(
            tab_hbm.at[idx_v.at[lax.rem(chunk, NIB)]],
            rows_v.at[buf], sem_g.at[buf])

    def gather_wait(chunk, buf):
        pltpu.make_async_copy(
            tab_hbm.at[idx_v.at[lax.rem(chunk, NIB)]],
            rows_v.at[buf], sem_g.at[buf]).wait()

    def wb_issue(chunk, buf):
        base = (chunk0 + chunk) * CH
        pltpu.async_copy(out_v.at[buf], out_flat.at[pl.ds(base, CH)],
                         sem_wb.at[buf])

    def wb_wait(chunk, buf):
        base = (chunk0 + chunk) * CH
        pltpu.make_async_copy(
            out_v.at[buf], out_flat.at[pl.ds(base, CH)],
            sem_wb.at[buf]).wait()

    for c in range(3):
        idx_issue(c)
    idx_wait(0)
    gather_issue(0, 0)
    idx_wait(1)
    gather_issue(1, 1)

    def chunk_body(i, _):
        b = lax.rem(i, 2)
        gather_wait(i, b)

        @pl.when(i >= 2)
        def _():
            wb_wait(i - 2, b)

        p0 = lax.rem((chunk0 + i) * CH, T)

        def row_body(r8, _):
            r0 = r8 * 8
            for rr in range(8):
                r = r0 + rr
                for d in range(VPR):
                    sl = pl.ds(d * LANES, LANES)
                    out_v[b, r, sl] = (
                        rows_v[b, r, sl] * SCALE + pe_v[p0 + r, sl])
            return ()

        lax.fori_loop(0, CH // 8, row_body, ())
        wb_issue(i, b)

        @pl.when(i + 3 < NCHUNK)
        def _():
            idx_issue(i + 3)

        @pl.when(i + 2 < NCHUNK)
        def _():
            idx_wait(i + 2)
            gather_issue(i + 2, b)

        return ()

    lax.fori_loop(0, NCHUNK, chunk_body, ())
    for j in range(NCHUNK - 2, NCHUNK):
        wb_wait(j, j % 2)


def kernel(x, tok_emb):
    pe = _pos_encoding()
    pe2 = jnp.concatenate([pe, pe[:CH]], axis=0)
    tabp = jnp.pad(tok_emb, ((0, 0), (0, DP - D)))
    x2 = x.reshape(B * T // CH, CH).astype(jnp.int32)
    return _emb_kernel(x2, tabp, pe2)
